# Initial kernel scaffold; baseline (speedup 1.0000x reference)
import jax
import jax.numpy as jnp
from jax.experimental import pallas as pl


def kernel(x, Wg, bg, W1, b1, W2, b2, ln_gamma, ln_beta, Wc, bc, Wp, bp):
    raise NotImplementedError("write your pallas kernel here")


# speedup vs baseline: 3.0151x; 3.0151x over previous
import functools

import jax
import jax.numpy as jnp
from jax.experimental import pallas as pl
from jax.experimental.pallas import tpu as pltpu

NE = 8
TK = 2
D = 768
NTOK = 2048
NS = NTOK * TK


def _gate_body(x_ref, wg_ref, bg_ref, e12_ref):
    logits = jnp.dot(x_ref[...], wg_ref[...],
                     preferred_element_type=jnp.float32) + bg_ref[...]
    iota = jax.lax.broadcasted_iota(jnp.int32, logits.shape, 1)
    top1 = jnp.argmax(logits, axis=1).astype(jnp.int32)
    masked = jnp.where(iota == top1[:, None], -jnp.inf, logits)
    top2 = jnp.argmax(masked, axis=1).astype(jnp.int32)
    e12_ref[...] = jnp.concatenate([top1[:, None], top2[:, None]], axis=1)


def _gate(x, Wg, bg):
    return pl.pallas_call(
        _gate_body,
        out_shape=jax.ShapeDtypeStruct((NTOK, TK), jnp.int32),
    )(x, Wg, bg.reshape(1, NE))


def _moe_body(e_ref, x_ref, w1_ref, b1_ref, w2_ref, b2_ref, g_ref, bt_ref,
              wc_ref, bc_ref, wp_ref, bp_ref, out_ref, conf_ref, prob_ref):
    e = pl.program_id(1)
    xb = x_ref[...]
    h = jnp.dot(xb, w1_ref[0], preferred_element_type=jnp.float32) + b1_ref[0]
    h = 0.5 * h * (1.0 + jax.lax.erf(h * 0.7071067811865476))
    h = jnp.dot(h, w2_ref[0], preferred_element_type=jnp.float32) + b2_ref[0]
    mu = jnp.mean(h, axis=1, keepdims=True)
    var = jnp.mean((h - mu) ** 2, axis=1, keepdims=True)
    hn = (h - mu) * jax.lax.rsqrt(var + 1e-5) * g_ref[0] + bt_ref[0]
    c = jax.nn.sigmoid(jnp.sum(hn * wc_ref[0], axis=1, keepdims=True)
                       + bc_ref[0])
    pr = jnp.dot(h, wp_ref[...], preferred_element_type=jnp.float32) + bp_ref[...]
    pr = jax.nn.softmax(pr, axis=1)

    @pl.when(e == 0)
    def _():
        out_ref[...] = jnp.zeros_like(out_ref)
        conf_ref[...] = jnp.zeros_like(conf_ref)
        prob_ref[...] = jnp.zeros_like(prob_ref)

    m = e_ref[...] == e
    out_ref[...] = jnp.where(m, h, out_ref[...])
    conf_ref[...] = jnp.where(m, c, conf_ref[...])
    prob_ref[...] = jnp.where(m, pr, prob_ref[...])


def kernel(x, Wg, bg, W1, b1, W2, b2, ln_gamma, ln_beta, Wc, bc, Wp, bp):
    e12 = _gate(x, Wg, bg)
    e_flat = e12.reshape(NS, 1)

    TROWS = 512
    ntiles = NS // TROWS
    out, conf, prob = pl.pallas_call(
        _moe_body,
        grid=(ntiles, NE),
        in_specs=[
            pl.BlockSpec((TROWS, 1), lambda r, e: (r, 0)),
            pl.BlockSpec((TROWS, D), lambda r, e: (r % (NTOK // TROWS), 0)),
            pl.BlockSpec((1, D, D), lambda r, e: (e, 0, 0)),
            pl.BlockSpec((1, 1, D), lambda r, e: (e, 0, 0)),
            pl.BlockSpec((1, D, D), lambda r, e: (e, 0, 0)),
            pl.BlockSpec((1, 1, D), lambda r, e: (e, 0, 0)),
            pl.BlockSpec((1, 1, D), lambda r, e: (e, 0, 0)),
            pl.BlockSpec((1, 1, D), lambda r, e: (e, 0, 0)),
            pl.BlockSpec((1, 1, D), lambda r, e: (e, 0, 0)),
            pl.BlockSpec((1, 1, 1), lambda r, e: (e, 0, 0)),
            pl.BlockSpec((D, TK), lambda r, e: (0, 0)),
            pl.BlockSpec((1, TK), lambda r, e: (0, 0)),
        ],
        out_specs=[
            pl.BlockSpec((TROWS, D), lambda r, e: (r, 0)),
            pl.BlockSpec((TROWS, 1), lambda r, e: (r, 0)),
            pl.BlockSpec((TROWS, TK), lambda r, e: (r, 0)),
        ],
        out_shape=[
            jax.ShapeDtypeStruct((NS, D), jnp.float32),
            jax.ShapeDtypeStruct((NS, 1), jnp.float32),
            jax.ShapeDtypeStruct((NS, TK), jnp.float32),
        ],
        compiler_params=pltpu.CompilerParams(
            dimension_semantics=("arbitrary", "arbitrary"),
        ),
    )(e_flat, x, W1, b1.reshape(NE, 1, D), W2, b2.reshape(NE, 1, D),
      ln_gamma.reshape(NE, 1, D), ln_beta.reshape(NE, 1, D),
      Wc.reshape(NE, 1, D), bc.reshape(NE, 1, 1), Wp, bp.reshape(1, TK))

    moe_out = out.reshape(NTOK, TK, D)
    conf_out = conf.reshape(NTOK, TK)
    prob_GT = prob.reshape(NTOK, TK, TK)
    return moe_out, conf_out, prob_GT
